# baseline (device time: 60983 ns/iter reference)
import jax
import jax.numpy as jnp
from jax import lax
from jax.experimental import pallas as pl
from jax.experimental.pallas import tpu as pltpu

N_DEV = 4


def kernel(x, w_mat, scale_x, scale_w):
    m_global, k_per = x.shape
    k_global, n = w_mat.shape
    m_per = m_global // N_DEV
    assert k_per * N_DEV == k_global

    x8 = x.astype(jnp.float8_e4m3fn)
    w8 = w_mat.astype(jnp.float8_e4m3fn)

    def body(x_ref, w_ref, sx_ref, sw_ref, out_ref, comm_ref, send_sems, recv_sems):
        my = lax.axis_index("i")

        barrier_sem = pltpu.get_barrier_semaphore()
        for d in range(1, N_DEV):
            pl.semaphore_signal(
                barrier_sem,
                inc=1,
                device_id=((my + d) % N_DEV,),
                device_id_type=pl.DeviceIdType.MESH,
            )
        pl.semaphore_wait(barrier_sem, N_DEV - 1)

        comm_ref[my] = x_ref[pl.ds(my * m_per, m_per), :]

        rdmas = []
        for d in range(1, N_DEV):
            dst = (my + d) % N_DEV
            rdma = pltpu.make_async_remote_copy(
                src_ref=x_ref.at[pl.ds(dst * m_per, m_per), :],
                dst_ref=comm_ref.at[my],
                send_sem=send_sems.at[d - 1],
                recv_sem=recv_sems.at[d - 1],
                device_id=(dst,),
                device_id_type=pl.DeviceIdType.MESH,
            )
            rdma.start()
            rdmas.append(rdma)
        for rdma in rdmas:
            rdma.wait()

        acc = jnp.zeros((m_per, n), jnp.float32)
        for j in range(N_DEV):
            acc = acc + jnp.dot(
                comm_ref[j],
                w_ref[j * k_per : (j + 1) * k_per, :],
                preferred_element_type=jnp.float32,
            )
        y = acc * (sx_ref[0] * sw_ref[0])
        out_ref[:, :] = y * jax.nn.sigmoid(y)

    return pl.pallas_call(
        body,
        out_shape=jax.ShapeDtypeStruct((m_per, n), jnp.float32),
        in_specs=[
            pl.BlockSpec(memory_space=pltpu.VMEM),
            pl.BlockSpec(memory_space=pltpu.VMEM),
            pl.BlockSpec(memory_space=pltpu.SMEM),
            pl.BlockSpec(memory_space=pltpu.SMEM),
        ],
        out_specs=pl.BlockSpec(memory_space=pltpu.VMEM),
        scratch_shapes=[
            pltpu.VMEM((N_DEV, m_per, k_per), jnp.float8_e4m3fn),
            pltpu.SemaphoreType.DMA((N_DEV - 1,)),
            pltpu.SemaphoreType.DMA((N_DEV - 1,)),
        ],
        compiler_params=pltpu.CompilerParams(collective_id=0),
    )(x8, w8, scale_x, scale_w)


# device time: 47718 ns/iter; 1.2780x vs baseline; 1.2780x over previous
import jax
import jax.numpy as jnp
from jax import lax
from jax.experimental import pallas as pl
from jax.experimental.pallas import tpu as pltpu

N_DEV = 4
F8 = jnp.float8_e4m3fn


def kernel(x, w_mat, scale_x, scale_w):
    m_global, k_per = x.shape
    k_global, n = w_mat.shape
    m_per = m_global // N_DEV
    assert k_per * N_DEV == k_global

    def body(
        x_ref,
        w_ref,
        sx_ref,
        sw_ref,
        out_ref,
        xs_ref,
        comm_ref,
        wf_ref,
        w8_ref,
        send_sems,
        recv_sems,
        wdma_sem,
    ):
        my = lax.axis_index("i")

        barrier_sem = pltpu.get_barrier_semaphore()
        for d in range(1, N_DEV):
            pl.semaphore_signal(
                barrier_sem,
                inc=1,
                device_id=((my + d) % N_DEV,),
                device_id_type=pl.DeviceIdType.MESH,
            )

        p1 = (my + 1) % N_DEV
        xs_ref[p1] = x_ref[pl.ds(p1 * m_per, m_per), :].astype(F8)

        pl.semaphore_wait(barrier_sem, N_DEV - 1)

        rdmas = {}
        for d in (1, 3, 2):
            dst = (my + d) % N_DEV
            if d != 1:
                xs_ref[dst] = x_ref[pl.ds(dst * m_per, m_per), :].astype(F8)
            rdma = pltpu.make_async_remote_copy(
                src_ref=xs_ref.at[dst],
                dst_ref=comm_ref.at[my],
                send_sem=send_sems.at[d - 1],
                recv_sem=recv_sems.at[d - 1],
                device_id=(dst,),
                device_id_type=pl.DeviceIdType.MESH,
            )
            rdma.start()
            rdmas[d] = rdma

        order = [my, (my - 1) % N_DEV, (my + 1) % N_DEV, (my + 2) % N_DEV]
        offs = [None, 1, 3, 2]

        def w_dma(t):
            return pltpu.make_async_copy(
                w_ref.at[pl.ds(order[t] * k_per, k_per), :],
                wf_ref,
                wdma_sem,
            )

        w_dma(0).start()

        for t in range(N_DEV):
            w_dma(t).wait()
            w8_ref[t % 2] = wf_ref[:, :].astype(F8)
            if t + 1 < N_DEV:
                w_dma(t + 1).start()
            if t == 0:
                a = x_ref[pl.ds(my * m_per, m_per), :].astype(F8)
            else:
                rdmas[offs[t]].wait_recv()
                a = comm_ref[order[t]]
            n_h = n // 2
            for h in range(2):
                partial = jnp.dot(
                    a,
                    w8_ref[t % 2, :, pl.ds(h * n_h, n_h)],
                    preferred_element_type=jnp.float32,
                )
                if t == 0:
                    out_ref[:, pl.ds(h * n_h, n_h)] = partial
                else:
                    out_ref[:, pl.ds(h * n_h, n_h)] = (
                        out_ref[:, pl.ds(h * n_h, n_h)] + partial
                    )

        for d in (1, 3, 2):
            rdmas[d].wait_send()

        s = sx_ref[0] * sw_ref[0]
        m_c = m_per // 4
        for c in range(4):
            y = out_ref[pl.ds(c * m_c, m_c), :] * s
            out_ref[pl.ds(c * m_c, m_c), :] = y * jax.nn.sigmoid(y)

    return pl.pallas_call(
        body,
        out_shape=jax.ShapeDtypeStruct((m_per, n), jnp.float32),
        in_specs=[
            pl.BlockSpec(memory_space=pltpu.VMEM),
            pl.BlockSpec(memory_space=pl.ANY),
            pl.BlockSpec(memory_space=pltpu.SMEM),
            pl.BlockSpec(memory_space=pltpu.SMEM),
        ],
        out_specs=pl.BlockSpec(memory_space=pltpu.VMEM),
        scratch_shapes=[
            pltpu.VMEM((N_DEV, m_per, k_per), F8),
            pltpu.VMEM((N_DEV, m_per, k_per), F8),
            pltpu.VMEM((k_per, n), jnp.float32),
            pltpu.VMEM((2, k_per, n), F8),
            pltpu.SemaphoreType.DMA((N_DEV - 1,)),
            pltpu.SemaphoreType.DMA((N_DEV - 1,)),
            pltpu.SemaphoreType.DMA,
        ],
        compiler_params=pltpu.CompilerParams(
            collective_id=0, vmem_limit_bytes=64 * 1024 * 1024
        ),
    )(x, w_mat, scale_x, scale_w)


# device time: 40353 ns/iter; 1.5112x vs baseline; 1.1825x over previous
import jax
import jax.numpy as jnp
from jax import lax
from jax.experimental import pallas as pl
from jax.experimental.pallas import tpu as pltpu

N_DEV = 4
F8 = jnp.float8_e4m3fn


def kernel(x, w_mat, scale_x, scale_w):
    m_global, k_per = x.shape
    k_global, n = w_mat.shape
    m_per = m_global // N_DEV
    assert k_per * N_DEV == k_global
    m_h = m_per // 2

    def body(
        x_ref,
        w_ref,
        sx_ref,
        sw_ref,
        out_ref,
        xf_ref,
        xs_ref,
        comm_ref,
        wf_ref,
        w8_ref,
        acc_ref,
        xdma_sems,
        send_sems,
        recv_sems,
        wdma_sem,
        odma_sems,
    ):
        my = lax.axis_index("i")

        barrier_sem = pltpu.get_barrier_semaphore()
        for d in range(1, N_DEV):
            pl.semaphore_signal(
                barrier_sem,
                inc=1,
                device_id=((my + d) % N_DEV,),
                device_id_type=pl.DeviceIdType.MESH,
            )

        send_order = [(my + 1) % N_DEV, (my + 3) % N_DEV, (my + 2) % N_DEV, my]
        send_offs = [1, 3, 2]
        loads = [(idx, half) for idx in range(N_DEV) for half in range(2)]

        def x_dma(load, slot):
            idx, half = load
            p = send_order[idx]
            return pltpu.make_async_copy(
                x_ref.at[pl.ds(p * m_per + half * m_h, m_h), :],
                xf_ref.at[slot],
                xdma_sems.at[slot],
            )

        x_dma(loads[0], 0).start()

        rdmas = {}
        for li, (idx, half) in enumerate(loads):
            if li + 1 < len(loads):
                x_dma(loads[li + 1], (li + 1) % 2).start()
            x_dma((idx, half), li % 2).wait()
            p = send_order[idx]
            xs_ref[p, half] = xf_ref[li % 2].astype(F8)
            if li == 0:
                pl.semaphore_wait(barrier_sem, N_DEV - 1)
            if idx < 3:
                d = send_offs[idx]
                rdma = pltpu.make_async_remote_copy(
                    src_ref=xs_ref.at[p, half],
                    dst_ref=comm_ref.at[my, half],
                    send_sem=send_sems.at[d - 1, half],
                    recv_sem=recv_sems.at[d - 1, half],
                    device_id=(p,),
                    device_id_type=pl.DeviceIdType.MESH,
                )
                rdma.start()
                rdmas[(d, half)] = rdma

        order = [my, (my - 1) % N_DEV, (my + 1) % N_DEV, (my + 2) % N_DEV]
        offs = [None, 1, 3, 2]

        def w_dma(t):
            return pltpu.make_async_copy(
                w_ref.at[pl.ds(order[t] * k_per, k_per), :],
                wf_ref,
                wdma_sem,
            )

        w_dma(0).start()

        s = sx_ref[0] * sw_ref[0]
        n_h = n // 2
        for t in range(N_DEV):
            w_dma(t).wait()
            w8_ref[t % 2] = wf_ref[:, :].astype(F8)
            if t + 1 < N_DEV:
                w_dma(t + 1).start()
            for half in range(2):
                if t == 0:
                    a = xs_ref[my, half]
                else:
                    rdmas[(offs[t], half)].wait_recv()
                    a = comm_ref[order[t], half]
                ds_m = pl.ds(half * m_h, m_h)
                for h in range(2):
                    ds_h = pl.ds(h * n_h, n_h)
                    partial = jnp.dot(
                        a,
                        w8_ref[t % 2, :, ds_h],
                        preferred_element_type=jnp.float32,
                    )
                    if t == 0:
                        acc_ref[h, ds_m, :] = partial
                    elif t < N_DEV - 1:
                        acc_ref[h, ds_m, :] = acc_ref[h, ds_m, :] + partial
                    else:
                        y = (acc_ref[h, ds_m, :] + partial) * s
                        acc_ref[h, ds_m, :] = y * jax.nn.sigmoid(y)
                        pltpu.make_async_copy(
                            acc_ref.at[h, ds_m, :],
                            out_ref.at[ds_m, ds_h],
                            odma_sems.at[h, half],
                        ).start()

        for key in rdmas:
            rdmas[key].wait_send()
        for h in range(2):
            for half in range(2):
                pltpu.make_async_copy(
                    acc_ref.at[h, pl.ds(half * m_h, m_h), :],
                    out_ref.at[pl.ds(half * m_h, m_h), pl.ds(h * n_h, n_h)],
                    odma_sems.at[h, half],
                ).wait()

    return pl.pallas_call(
        body,
        out_shape=jax.ShapeDtypeStruct((m_per, n), jnp.float32),
        in_specs=[
            pl.BlockSpec(memory_space=pl.ANY),
            pl.BlockSpec(memory_space=pl.ANY),
            pl.BlockSpec(memory_space=pltpu.SMEM),
            pl.BlockSpec(memory_space=pltpu.SMEM),
        ],
        out_specs=pl.BlockSpec(memory_space=pl.ANY),
        scratch_shapes=[
            pltpu.VMEM((2, m_h, k_per), jnp.float32),
            pltpu.VMEM((N_DEV, 2, m_h, k_per), F8),
            pltpu.VMEM((N_DEV, 2, m_h, k_per), F8),
            pltpu.VMEM((k_per, n), jnp.float32),
            pltpu.VMEM((2, k_per, n), F8),
            pltpu.VMEM((2, m_per, n // 2), jnp.float32),
            pltpu.SemaphoreType.DMA((2,)),
            pltpu.SemaphoreType.DMA((N_DEV - 1, 2)),
            pltpu.SemaphoreType.DMA((N_DEV - 1, 2)),
            pltpu.SemaphoreType.DMA,
            pltpu.SemaphoreType.DMA((2, 2)),
        ],
        compiler_params=pltpu.CompilerParams(
            collective_id=0,
            vmem_limit_bytes=64 * 1024 * 1024,
        ),
    )(x, w_mat, scale_x, scale_w)
